# baseline (speedup 1.0000x reference)
import jax
import jax.numpy as jnp
from jax.experimental import pallas as pl


_BLOCK_ROWS = 2048


def _add_kernel(x_ref, pos_ref, out_ref):
    out_ref[...] = x_ref[...] + pos_ref[...]


def kernel(x, pos_table):
    batch, seq_len, d_model = x.shape
    xf = x.reshape(batch * seq_len, d_model)
    n_blocks = (batch * seq_len) // _BLOCK_ROWS
    s_blocks = seq_len // _BLOCK_ROWS

    out = pl.pallas_call(
        _add_kernel,
        grid=(n_blocks,),
        in_specs=[
            pl.BlockSpec((_BLOCK_ROWS, d_model), lambda i: (i, 0)),
            pl.BlockSpec((_BLOCK_ROWS, d_model), lambda i: (i % s_blocks, 0)),
        ],
        out_specs=pl.BlockSpec((_BLOCK_ROWS, d_model), lambda i: (i, 0)),
        out_shape=jax.ShapeDtypeStruct((batch * seq_len, d_model), x.dtype),
    )(xf, pos_table)
    return out.reshape(batch, seq_len, d_model)


# speedup vs baseline: 1.2689x; 1.2689x over previous
import jax
import jax.numpy as jnp
from jax.experimental import pallas as pl


_BLOCK_ROWS = 1024


def _add_kernel(x_ref, pos_ref, out_ref):
    out_ref[0] = x_ref[0] + pos_ref[...]


def kernel(x, pos_table):
    batch, seq_len, d_model = x.shape
    s_blocks = seq_len // _BLOCK_ROWS

    return pl.pallas_call(
        _add_kernel,
        grid=(s_blocks, batch),
        in_specs=[
            pl.BlockSpec((1, _BLOCK_ROWS, d_model), lambda s, b: (b, s, 0)),
            pl.BlockSpec((_BLOCK_ROWS, d_model), lambda s, b: (s, 0)),
        ],
        out_specs=pl.BlockSpec((1, _BLOCK_ROWS, d_model), lambda s, b: (b, s, 0)),
        out_shape=jax.ShapeDtypeStruct((batch, seq_len, d_model), x.dtype),
    )(x, pos_table)
